# baseline (speedup 1.0000x reference)
import jax
import jax.numpy as jnp
from jax import lax
from jax.experimental import pallas as pl
from jax.experimental.pallas import tpu as pltpu
from jax.experimental.pallas import tpu_sc as plsc

_B, _N = 1024, 65536
_K = 32
_STRIDE = _N // _K
_LANES = 16


def _sc_body(x_hbm, out_hbm, buf, out_v, sem):
    nc = plsc.get_sparse_core_info().num_cores
    wid = lax.axis_index("s") * nc + lax.axis_index("c")
    rows = _B // (nc * 16)
    r0 = wid * rows

    col_lane = lax.iota(jnp.int32, _LANES)
    zeros = jnp.zeros((_LANES,), jnp.int32)

    for h in range(_K // _LANES):
        copies = [
            pltpu.make_async_copy(
                x_hbm.at[pl.ds(r0, rows), pl.ds((h * _LANES + j) * _STRIDE, 128)],
                buf.at[j],
                sem,
            )
            for j in range(_LANES)
        ]
        for cp in copies:
            cp.start()
        for cp in copies:
            cp.wait()

        def extract(b, carry):
            vals = plsc.load_gather(
                buf, [col_lane, jnp.full((_LANES,), b, jnp.int32), zeros]
            )
            out_v[b, pl.ds(h * _LANES, _LANES)] = vals
            return carry

        lax.fori_loop(0, rows, extract, 0)

    pltpu.sync_copy(out_v, out_hbm.at[pl.ds(r0, rows), :])


def kernel(x):
    mesh = plsc.VectorSubcoreMesh(core_axis_name="c", subcore_axis_name="s")
    return pl.kernel(
        _sc_body,
        mesh=mesh,
        out_type=jax.ShapeDtypeStruct((_B, _K), jnp.float32),
        scratch_types=[
            pltpu.VMEM((_LANES, _B // 32, 128), jnp.float32),
            pltpu.VMEM((_B // 32, _K), jnp.float32),
            pltpu.SemaphoreType.DMA,
        ],
        compiler_params=pltpu.CompilerParams(needs_layout_passes=False),
    )(x)
